# baseline (speedup 1.0000x reference)
import jax
import jax.numpy as jnp
from jax.experimental import pallas as pl
from jax.experimental.pallas import tpu as pltpu

B, D, STRIPE, NS, K = 4096, 2048, 128, 32, 8
H = NS * STRIPE
BT = 512
NB = B // BT


def _mask_into(mexp_ref, slot, xref, wr_ref, br_ref):
    scores = jnp.dot(xref[...], wr_ref[...],
                     preferred_element_type=jnp.float32)
    scores = scores + br_ref[...]
    cur = scores
    for _ in range(K - 1):
        m = jnp.max(cur, axis=1, keepdims=True)
        cur = jnp.where(cur == m, -jnp.inf, cur)
    thr = jnp.max(cur, axis=1, keepdims=True)
    mexp_ref[slot] = (scores >= thr).astype(jnp.bfloat16)


def _body(xb_ref, xb2_ref, be_ref, bd_ref, br_ref,
          wr_hbm, rsel_hbm, we_hbm, wd_hbm, out_hbm,
          we_v, wd_v, wr_v, rsel_v, mexp_ref, code_ref, out_v,
          sem_e, sem_d, sem_w, sem_o):
    i = pl.program_id(0)

    @pl.when(i > 0)
    def _():
        pltpu.make_async_copy(
            out_v, out_hbm.at[pl.ds((i - 1) * BT, BT), :], sem_o).wait()

    @pl.when(i == 0)
    def _():
        pltpu.make_async_copy(we_hbm, we_v, sem_e).start()
        pltpu.make_async_copy(wd_hbm, wd_v, sem_d).start()
        pltpu.make_async_copy(wr_hbm, wr_v, sem_w).start()
        pltpu.make_async_copy(rsel_hbm, rsel_v, sem_w).start()
        pltpu.make_async_copy(wr_hbm, wr_v, sem_w).wait()
        pltpu.make_async_copy(rsel_hbm, rsel_v, sem_w).wait()
        _mask_into(mexp_ref, 0, xb_ref, wr_v, br_ref)

    @pl.when(i < NB - 1)
    def _():
        _mask_into(mexp_ref, (i + 1) % 2, xb2_ref, wr_v, br_ref)

    @pl.when(i == 0)
    def _():
        pltpu.make_async_copy(we_hbm, we_v, sem_e).wait()

    for half in range(2):
        sl = slice(half * (H // 2), (half + 1) * (H // 2))
        mexp = jnp.dot(mexp_ref[i % 2], rsel_v[:, sl],
                       preferred_element_type=jnp.float32)
        e = jnp.dot(xb_ref[...], we_v[:, sl],
                    preferred_element_type=jnp.float32)
        e = jnp.maximum(e + be_ref[:, sl], 0.0) * mexp
        code_ref[:, sl] = e.astype(jnp.bfloat16)

    @pl.when(i == 0)
    def _():
        pltpu.make_async_copy(wd_hbm, wd_v, sem_d).wait()

    part = jnp.dot(code_ref[...], wd_v[...],
                   preferred_element_type=jnp.float32)
    out_v[...] = jnp.maximum(part + bd_ref[...], 0.0)
    cp_o = pltpu.make_async_copy(
        out_v, out_hbm.at[pl.ds(i * BT, BT), :], sem_o)
    cp_o.start()

    @pl.when(i == NB - 1)
    def _():
        cp_o.wait()


def _run(xb, we, be2, wd, bd2, wr, br2, interpret=False):
    rsel = (jnp.arange(NS)[:, None] ==
            (jnp.arange(H) // STRIPE)[None, :]).astype(jnp.bfloat16)
    grid = (NB,)
    return pl.pallas_call(
        _body,
        grid=grid,
        in_specs=[
            pl.BlockSpec((BT, D), lambda i: (i, 0)),
            pl.BlockSpec((BT, D), lambda i: (jnp.minimum(i + 1, NB - 1), 0)),
            pl.BlockSpec((1, H), lambda i: (0, 0)),
            pl.BlockSpec((1, D), lambda i: (0, 0)),
            pl.BlockSpec((1, NS), lambda i: (0, 0)),
            pl.BlockSpec(memory_space=pl.ANY),
            pl.BlockSpec(memory_space=pl.ANY),
            pl.BlockSpec(memory_space=pl.ANY),
            pl.BlockSpec(memory_space=pl.ANY),
        ],
        out_specs=pl.BlockSpec(memory_space=pl.ANY),
        out_shape=jax.ShapeDtypeStruct((B, D), jnp.float32),
        scratch_shapes=[
            pltpu.VMEM((D, H), jnp.bfloat16),
            pltpu.VMEM((H, D), jnp.bfloat16),
            pltpu.VMEM((D, NS), jnp.bfloat16),
            pltpu.VMEM((NS, H), jnp.bfloat16),
            pltpu.VMEM((2, BT, NS), jnp.bfloat16),
            pltpu.VMEM((BT, H), jnp.bfloat16),
            pltpu.VMEM((BT, D), jnp.float32),
            pltpu.SemaphoreType.DMA,
            pltpu.SemaphoreType.DMA,
            pltpu.SemaphoreType.DMA,
            pltpu.SemaphoreType.DMA,
        ],
        compiler_params=pltpu.CompilerParams(
            dimension_semantics=("arbitrary",),
        ),
        interpret=interpret,
    )(xb, xb, be2, bd2, br2, wr, rsel, we, wd)


def kernel(x, W_enc, b_enc, W_dec, b_dec, W_rout, b_rout):
    xb = x.astype(jnp.bfloat16)
    we = W_enc.astype(jnp.bfloat16)
    wd = W_dec.astype(jnp.bfloat16)
    wr = W_rout.astype(jnp.bfloat16)
    be2 = b_enc.reshape(1, H)
    bd2 = b_dec.reshape(1, D)
    br2 = b_rout.reshape(1, NS)
    return _run(xb, we, be2, wd, bd2, wr, br2)


# speedup vs baseline: 1.0544x; 1.0544x over previous
import jax
import jax.numpy as jnp
from jax.experimental import pallas as pl
from jax.experimental.pallas import tpu as pltpu

B, D, STRIPE, NS, K = 4096, 2048, 128, 32, 8
H = NS * STRIPE
BT = 512
NB = B // BT


def _body(xb_ref, be_ref, bd_ref, br_ref,
          wr_hbm, rsel_hbm, we_hbm, wd_hbm, out_ref,
          we_v, wd_v, wr_v, rsel_v, sem_e, sem_d, sem_w):
    i = pl.program_id(0)

    @pl.when(i == 0)
    def _():
        pltpu.make_async_copy(we_hbm, we_v, sem_e).start()
        pltpu.make_async_copy(wd_hbm, wd_v, sem_d).start()
        pltpu.make_async_copy(wr_hbm, wr_v, sem_w).start()
        pltpu.make_async_copy(rsel_hbm, rsel_v, sem_w).start()
        pltpu.make_async_copy(wr_hbm, wr_v, sem_w).wait()
        pltpu.make_async_copy(rsel_hbm, rsel_v, sem_w).wait()

    scores = jnp.dot(xb_ref[...], wr_v[...],
                     preferred_element_type=jnp.float32)
    scores = scores + br_ref[...]
    st = scores.T
    cur = st
    for _ in range(K - 1):
        m = jnp.max(cur, axis=0, keepdims=True)
        cur = jnp.where(cur == m, -jnp.inf, cur)
    thr = jnp.max(cur, axis=0, keepdims=True)
    maskb = (st >= thr).astype(jnp.bfloat16)

    mexp = jax.lax.dot_general(maskb, rsel_v[...], (((0,), (0,)), ((), ())),
                               preferred_element_type=jnp.float32)

    @pl.when(i == 0)
    def _():
        pltpu.make_async_copy(we_hbm, we_v, sem_e).wait()

    e = jnp.dot(xb_ref[...], we_v[...], preferred_element_type=jnp.float32)
    e = jnp.maximum(e + be_ref[...], 0.0) * mexp
    code = e.astype(jnp.bfloat16)

    @pl.when(i == 0)
    def _():
        pltpu.make_async_copy(wd_hbm, wd_v, sem_d).wait()

    part = jnp.dot(code, wd_v[...], preferred_element_type=jnp.float32)
    out_ref[...] = jnp.maximum(part + bd_ref[...], 0.0)


def _run(xb, we, be2, wd, bd2, wr, br2, interpret=False):
    rsel = (jnp.arange(NS)[:, None] ==
            (jnp.arange(H) // STRIPE)[None, :]).astype(jnp.bfloat16)
    grid = (NB,)
    return pl.pallas_call(
        _body,
        grid=grid,
        in_specs=[
            pl.BlockSpec((BT, D), lambda i: (i, 0)),
            pl.BlockSpec((1, H), lambda i: (0, 0)),
            pl.BlockSpec((1, D), lambda i: (0, 0)),
            pl.BlockSpec((1, NS), lambda i: (0, 0)),
            pl.BlockSpec(memory_space=pl.ANY),
            pl.BlockSpec(memory_space=pl.ANY),
            pl.BlockSpec(memory_space=pl.ANY),
            pl.BlockSpec(memory_space=pl.ANY),
        ],
        out_specs=pl.BlockSpec((BT, D), lambda i: (i, 0)),
        out_shape=jax.ShapeDtypeStruct((B, D), jnp.float32),
        scratch_shapes=[
            pltpu.VMEM((D, H), jnp.bfloat16),
            pltpu.VMEM((H, D), jnp.bfloat16),
            pltpu.VMEM((D, NS), jnp.bfloat16),
            pltpu.VMEM((NS, H), jnp.bfloat16),
            pltpu.SemaphoreType.DMA,
            pltpu.SemaphoreType.DMA,
            pltpu.SemaphoreType.DMA,
        ],
        compiler_params=pltpu.CompilerParams(
            dimension_semantics=("arbitrary",),
        ),
        interpret=interpret,
    )(xb, be2, bd2, br2, wr, rsel, we, wd)


def kernel(x, W_enc, b_enc, W_dec, b_dec, W_rout, b_rout):
    xb = x.astype(jnp.bfloat16)
    we = W_enc.astype(jnp.bfloat16)
    wd = W_dec.astype(jnp.bfloat16)
    wr = W_rout.astype(jnp.bfloat16)
    be2 = b_enc.reshape(1, H)
    bd2 = b_dec.reshape(1, D)
    br2 = b_rout.reshape(1, NS)
    return _run(xb, we, be2, wd, bd2, wr, br2)


# speedup vs baseline: 1.0598x; 1.0051x over previous
import jax
import jax.numpy as jnp
from jax.experimental import pallas as pl
from jax.experimental.pallas import tpu as pltpu

B, D, STRIPE, NS, K = 4096, 2048, 128, 32, 8
H = NS * STRIPE
BT = 512
NB = B // BT


def _body(xb_ref, be_ref, bd_ref, br_ref,
          wr_hbm, rsel_hbm, we_hbm, wd_hbm, out_ref,
          we_v, wd_v, wr_v, rsel_v, sem_e, sem_d, sem_w):
    i = pl.program_id(0)

    @pl.when(i == 0)
    def _():
        pltpu.make_async_copy(we_hbm, we_v, sem_e).start()
        pltpu.make_async_copy(wd_hbm, wd_v, sem_d).start()
        pltpu.make_async_copy(wr_hbm, wr_v, sem_w).start()
        pltpu.make_async_copy(rsel_hbm, rsel_v, sem_w).start()
        pltpu.make_async_copy(wr_hbm, wr_v, sem_w).wait()
        pltpu.make_async_copy(rsel_hbm, rsel_v, sem_w).wait()

    scores = jnp.dot(xb_ref[...], wr_v[...],
                     preferred_element_type=jnp.float32)
    scores = scores + br_ref[...]
    st = scores.T
    cur = st
    for _ in range(K - 1):
        m = jnp.max(cur, axis=0, keepdims=True)
        cur = jnp.where(cur == m, -jnp.inf, cur)
    thr = jnp.max(cur, axis=0, keepdims=True)
    maskb = (st >= thr).astype(jnp.bfloat16)

    mexp = jax.lax.dot_general(maskb, rsel_v[...], (((0,), (0,)), ((), ())),
                               preferred_element_type=jnp.float32)

    @pl.when(i == 0)
    def _():
        pltpu.make_async_copy(we_hbm, we_v, sem_e).wait()
        pltpu.make_async_copy(wd_hbm, wd_v, sem_d).wait()

    H2 = H // 2
    e0 = jnp.dot(xb_ref[...], we_v[:, :H2], preferred_element_type=jnp.float32)
    e0 = jnp.maximum(e0 + be_ref[:, :H2], 0.0) * mexp[:, :H2]
    code0 = e0.astype(jnp.bfloat16)

    e1 = jnp.dot(xb_ref[...], we_v[:, H2:], preferred_element_type=jnp.float32)
    part0 = jnp.dot(code0, wd_v[:H2, :], preferred_element_type=jnp.float32)
    e1 = jnp.maximum(e1 + be_ref[:, H2:], 0.0) * mexp[:, H2:]
    code1 = e1.astype(jnp.bfloat16)

    part1 = jnp.dot(code1, wd_v[H2:, :], preferred_element_type=jnp.float32)
    out_ref[...] = jnp.maximum(part0 + part1 + bd_ref[...], 0.0)


def _run(xb, we, be2, wd, bd2, wr, br2, interpret=False):
    rsel = (jnp.arange(NS)[:, None] ==
            (jnp.arange(H) // STRIPE)[None, :]).astype(jnp.bfloat16)
    grid = (NB,)
    return pl.pallas_call(
        _body,
        grid=grid,
        in_specs=[
            pl.BlockSpec((BT, D), lambda i: (i, 0)),
            pl.BlockSpec((1, H), lambda i: (0, 0)),
            pl.BlockSpec((1, D), lambda i: (0, 0)),
            pl.BlockSpec((1, NS), lambda i: (0, 0)),
            pl.BlockSpec(memory_space=pl.ANY),
            pl.BlockSpec(memory_space=pl.ANY),
            pl.BlockSpec(memory_space=pl.ANY),
            pl.BlockSpec(memory_space=pl.ANY),
        ],
        out_specs=pl.BlockSpec((BT, D), lambda i: (i, 0)),
        out_shape=jax.ShapeDtypeStruct((B, D), jnp.float32),
        scratch_shapes=[
            pltpu.VMEM((D, H), jnp.bfloat16),
            pltpu.VMEM((H, D), jnp.bfloat16),
            pltpu.VMEM((D, NS), jnp.bfloat16),
            pltpu.VMEM((NS, H), jnp.bfloat16),
            pltpu.SemaphoreType.DMA,
            pltpu.SemaphoreType.DMA,
            pltpu.SemaphoreType.DMA,
        ],
        compiler_params=pltpu.CompilerParams(
            dimension_semantics=("arbitrary",),
        ),
        interpret=interpret,
    )(xb, be2, bd2, br2, wr, rsel, we, wd)


def kernel(x, W_enc, b_enc, W_dec, b_dec, W_rout, b_rout):
    xb = x.astype(jnp.bfloat16)
    we = W_enc.astype(jnp.bfloat16)
    wd = W_dec.astype(jnp.bfloat16)
    wr = W_rout.astype(jnp.bfloat16)
    be2 = b_enc.reshape(1, H)
    bd2 = b_dec.reshape(1, D)
    br2 = b_rout.reshape(1, NS)
    return _run(xb, we, be2, wd, bd2, wr, br2)


# speedup vs baseline: 1.0732x; 1.0126x over previous
import jax
import jax.numpy as jnp
from jax.experimental import pallas as pl
from jax.experimental.pallas import tpu as pltpu

B, D, STRIPE, NS, K = 4096, 2048, 128, 32, 8
H = NS * STRIPE
BT = 512


def _body(xb_ref, we_hbm, be_ref, wd_hbm, bd_ref, wr_ref, br_ref,
          out_ref, we_v, wd_v, mexp_ref, sem_e, sem_d):
    i = pl.program_id(0)

    @pl.when(i == 0)
    def _():
        pltpu.make_async_copy(we_hbm, we_v, sem_e).start()
        pltpu.make_async_copy(wd_hbm, wd_v, sem_d).start()

    scores = jnp.dot(xb_ref[...], wr_ref[...],
                     preferred_element_type=jnp.float32)
    scores = scores + br_ref[...]
    cur = scores
    for _ in range(K - 1):
        m = jnp.max(cur, axis=1, keepdims=True)
        cur = jnp.where(cur == m, -jnp.inf, cur)
    thr = jnp.max(cur, axis=1, keepdims=True)
    maskb = (scores >= thr).astype(jnp.bfloat16)
    rows = jax.lax.broadcasted_iota(jnp.int32, (NS, H), 0)
    cols = jax.lax.broadcasted_iota(jnp.int32, (NS, H), 1)
    r = (rows == cols // STRIPE).astype(jnp.bfloat16)
    mexp_ref[...] = jnp.dot(maskb, r,
                            preferred_element_type=jnp.float32
                            ).astype(jnp.bfloat16)

    @pl.when(i == 0)
    def _():
        pltpu.make_async_copy(we_hbm, we_v, sem_e).wait()

    e = jnp.dot(xb_ref[...], we_v[...], preferred_element_type=jnp.float32)
    e = jnp.maximum(e + be_ref[...], 0.0) * mexp_ref[...].astype(jnp.float32)
    code = e.astype(jnp.bfloat16)

    @pl.when(i == 0)
    def _():
        pltpu.make_async_copy(wd_hbm, wd_v, sem_d).wait()

    part = jnp.dot(code, wd_v[...], preferred_element_type=jnp.float32)
    out_ref[...] = jnp.maximum(part + bd_ref[...], 0.0)


def _run(xb, we, be2, wd, bd2, wr, br2, interpret=False):
    grid = (B // BT,)
    return pl.pallas_call(
        _body,
        grid=grid,
        in_specs=[
            pl.BlockSpec((BT, D), lambda i: (i, 0)),
            pl.BlockSpec(memory_space=pl.ANY),
            pl.BlockSpec((1, H), lambda i: (0, 0)),
            pl.BlockSpec(memory_space=pl.ANY),
            pl.BlockSpec((1, D), lambda i: (0, 0)),
            pl.BlockSpec((D, NS), lambda i: (0, 0)),
            pl.BlockSpec((1, NS), lambda i: (0, 0)),
        ],
        out_specs=pl.BlockSpec((BT, D), lambda i: (i, 0)),
        out_shape=jax.ShapeDtypeStruct((B, D), jnp.float32),
        scratch_shapes=[
            pltpu.VMEM((D, H), jnp.bfloat16),
            pltpu.VMEM((H, D), jnp.bfloat16),
            pltpu.VMEM((BT, H), jnp.bfloat16),
            pltpu.SemaphoreType.DMA,
            pltpu.SemaphoreType.DMA,
        ],
        compiler_params=pltpu.CompilerParams(
            dimension_semantics=("arbitrary",),
        ),
        interpret=interpret,
    )(xb, we, be2, wd, bd2, wr, br2)


def kernel(x, W_enc, b_enc, W_dec, b_dec, W_rout, b_rout):
    xb = x.astype(jnp.bfloat16)
    we = W_enc.astype(jnp.bfloat16)
    wd = W_dec.astype(jnp.bfloat16)
    wr = W_rout.astype(jnp.bfloat16)
    be2 = b_enc.reshape(1, H)
    bd2 = b_dec.reshape(1, D)
    br2 = b_rout.reshape(1, NS)
    return _run(xb, we, be2, wd, bd2, wr, br2)


# speedup vs baseline: 1.0794x; 1.0058x over previous
import jax
import jax.numpy as jnp
from jax.experimental import pallas as pl
from jax.experimental.pallas import tpu as pltpu

B, D, STRIPE, NS, K = 4096, 2048, 128, 32, 8
H = NS * STRIPE
BT = 512


def _body(xb_ref, we_hbm, be_ref, wd_hbm, bd_ref, wr_ref, br_ref,
          out_ref, we_v, wd_v, sem_e, sem_d):
    i = pl.program_id(0)

    @pl.when(i == 0)
    def _():
        pltpu.make_async_copy(we_hbm, we_v, sem_e).start()
        pltpu.make_async_copy(wd_hbm, wd_v, sem_d).start()

    scores = jnp.dot(xb_ref[...], wr_ref[...],
                     preferred_element_type=jnp.float32)
    scores = scores + br_ref[...]
    cur = scores
    for _ in range(K - 1):
        m = jnp.max(cur, axis=1, keepdims=True)
        cur = jnp.where(cur == m, -jnp.inf, cur)
    thr = jnp.max(cur, axis=1, keepdims=True)
    maskb = (scores >= thr).astype(jnp.bfloat16)
    rows = jax.lax.broadcasted_iota(jnp.int32, (NS, H), 0)
    cols = jax.lax.broadcasted_iota(jnp.int32, (NS, H), 1)
    r = (rows == cols // STRIPE).astype(jnp.bfloat16)
    mexp = jnp.dot(maskb, r, preferred_element_type=jnp.float32)

    @pl.when(i == 0)
    def _():
        pltpu.make_async_copy(we_hbm, we_v, sem_e).wait()

    e = jnp.dot(xb_ref[...], we_v[...], preferred_element_type=jnp.float32)
    e = jnp.maximum(e + be_ref[...], 0.0) * mexp
    code = e.astype(jnp.bfloat16)

    @pl.when(i == 0)
    def _():
        pltpu.make_async_copy(wd_hbm, wd_v, sem_d).wait()

    part = jnp.dot(code, wd_v[...], preferred_element_type=jnp.float32)
    out_ref[...] = jnp.maximum(part + bd_ref[...], 0.0)


def _run(xb, we, be2, wd, bd2, wr, br2, interpret=False):
    grid = (B // BT,)
    return pl.pallas_call(
        _body,
        grid=grid,
        in_specs=[
            pl.BlockSpec((BT, D), lambda i: (i, 0)),
            pl.BlockSpec(memory_space=pl.ANY),
            pl.BlockSpec((1, H), lambda i: (0, 0)),
            pl.BlockSpec(memory_space=pl.ANY),
            pl.BlockSpec((1, D), lambda i: (0, 0)),
            pl.BlockSpec((D, NS), lambda i: (0, 0)),
            pl.BlockSpec((1, NS), lambda i: (0, 0)),
        ],
        out_specs=pl.BlockSpec((BT, D), lambda i: (i, 0)),
        out_shape=jax.ShapeDtypeStruct((B, D), jnp.float32),
        scratch_shapes=[
            pltpu.VMEM((D, H), jnp.bfloat16),
            pltpu.VMEM((H, D), jnp.bfloat16),
            pltpu.SemaphoreType.DMA,
            pltpu.SemaphoreType.DMA,
        ],
        compiler_params=pltpu.CompilerParams(
            dimension_semantics=("arbitrary",),
        ),
        interpret=interpret,
    )(xb, we, be2, wd, bd2, wr, br2)


def kernel(x, W_enc, b_enc, W_dec, b_dec, W_rout, b_rout):
    xb = x.astype(jnp.bfloat16)
    we = W_enc.astype(jnp.bfloat16)
    wd = W_dec.astype(jnp.bfloat16)
    wr = W_rout.astype(jnp.bfloat16)
    be2 = b_enc.reshape(1, H)
    bd2 = b_dec.reshape(1, D)
    br2 = b_rout.reshape(1, NS)
    return _run(xb, we, be2, wd, bd2, wr, br2)
